# Initial kernel scaffold; baseline (speedup 1.0000x reference)
import jax
import jax.numpy as jnp
from jax.experimental import pallas as pl


def kernel(content_embeddings, type_ids, time_ids, event_table, time_table):
    raise NotImplementedError("write your pallas kernel here")


# speedup vs baseline: 2.9202x; 2.9202x over previous
import functools

import jax
import jax.numpy as jnp
from jax import lax
from jax.experimental import pallas as pl
from jax.experimental.pallas import tpu as pltpu
from jax.experimental.pallas import tpu_sc as plsc

L = 204800
H = 128

NC = 2
NS = 16
NW = NC * NS
RPW = L // NW
CB = 128
NCHUNK = RPW // CB
NCOL = H // 16


def _body(c_hbm, tid_hbm, mid_hbm, et_hbm, tt_hbm, out_hbm,
          tid_v, mid_v, c_v, trow_v, mrow_v, sem_t, sem_m, sem_c):
    wid = lax.axis_index("s") * NC + lax.axis_index("c")
    base0 = wid * RPW

    def step_fn(step, carry):
        base = base0 + step * CB
        pltpu.sync_copy(tid_hbm.at[pl.ds(base, CB)], tid_v)
        pltpu.sync_copy(mid_hbm.at[pl.ds(base, CB)], mid_v)
        gt = pltpu.async_copy(et_hbm.at[tid_v], trow_v, sem_t)
        gm = pltpu.async_copy(tt_hbm.at[mid_v], mrow_v, sem_m)
        gc = pltpu.async_copy(c_hbm.at[pl.ds(base, CB)], c_v, sem_c)
        gt.wait()
        gm.wait()
        gc.wait()

        def row_fn(r, rc):
            for j in range(NCOL):
                sl = pl.ds(j * 16, 16)
                c_v[r, sl] = c_v[r, sl] + trow_v[r, sl] + mrow_v[r, sl]
            return rc

        lax.fori_loop(0, CB, row_fn, 0)

        @pl.when((wid == 0) & (step == 0))
        def _fix_row0():
            for j in range(NCOL):
                sl = pl.ds(j * 16, 16)
                c_v[0, sl] = c_v[0, sl] - mrow_v[0, sl]

        pltpu.sync_copy(c_v, out_hbm.at[pl.ds(base, CB)])
        return carry

    lax.fori_loop(0, NCHUNK, step_fn, 0)


@jax.jit
def _encode(content_embeddings, type_ids, time_ids, event_table, time_table):
    mesh = plsc.VectorSubcoreMesh(core_axis_name="c", subcore_axis_name="s")
    run = functools.partial(
        pl.kernel,
        mesh=mesh,
        out_type=jax.ShapeDtypeStruct((L, H), jnp.float32),
        scratch_types=[
            pltpu.VMEM((CB,), jnp.int32),
            pltpu.VMEM((CB,), jnp.int32),
            pltpu.VMEM((CB, H), jnp.float32),
            pltpu.VMEM((CB, H), jnp.float32),
            pltpu.VMEM((CB, H), jnp.float32),
            pltpu.SemaphoreType.DMA,
            pltpu.SemaphoreType.DMA,
            pltpu.SemaphoreType.DMA,
        ],
    )(_body)
    return run(content_embeddings, type_ids, time_ids, event_table, time_table)


def kernel(content_embeddings, type_ids, time_ids, event_table, time_table):
    return _encode(
        content_embeddings,
        type_ids.astype(jnp.int32),
        time_ids.astype(jnp.int32),
        event_table,
        time_table,
    )


# speedup vs baseline: 2.9649x; 1.0153x over previous
import functools

import jax
import jax.numpy as jnp
from jax import lax
from jax.experimental import pallas as pl
from jax.experimental.pallas import tpu as pltpu
from jax.experimental.pallas import tpu_sc as plsc

L = 204800
H = 128

NC = 2
NS = 16
NW = NC * NS
RPW = L // NW
CB = 128
NCHUNK = RPW // CB
NCOL = H // 16


def _body(c_hbm, tid_hbm, mid_hbm, et_hbm, tt_hbm, out_hbm,
          tid0, tid1, mid0, mid1, cv0, cv1, tv0, tv1, mv0, mv1,
          st0, st1, sm0, sm1, sc0, sc1, so0, so1, si0, si1):
    tid_v = (tid0, tid1)
    mid_v = (mid0, mid1)
    c_v = (cv0, cv1)
    t_v = (tv0, tv1)
    m_v = (mv0, mv1)
    sem_t = (st0, st1)
    sem_m = (sm0, sm1)
    sem_c = (sc0, sc1)
    sem_o = (so0, so1)
    sem_i = (si0, si1)

    wid = lax.axis_index("s") * NC + lax.axis_index("c")
    base0 = wid * RPW

    def idx_slice(n):
        return pl.ds(base0 + n * CB, CB)

    pltpu.sync_copy(tid_hbm.at[idx_slice(0)], tid_v[0])
    pltpu.sync_copy(mid_hbm.at[idx_slice(0)], mid_v[0])
    pltpu.sync_copy(tid_hbm.at[idx_slice(1)], tid_v[1])
    pltpu.sync_copy(mid_hbm.at[idx_slice(1)], mid_v[1])
    pltpu.async_copy(et_hbm.at[tid_v[0]], t_v[0], sem_t[0])
    pltpu.async_copy(tt_hbm.at[mid_v[0]], m_v[0], sem_m[0])
    pltpu.async_copy(c_hbm.at[idx_slice(0)], c_v[0], sem_c[0])

    def pair_fn(g, carry):
        for b in (0, 1):
            nb = 1 - b
            n = 2 * g + b

            pltpu.make_async_copy(et_hbm.at[tid_v[b]], t_v[b], sem_t[b]).wait()
            pltpu.make_async_copy(tt_hbm.at[mid_v[b]], m_v[b], sem_m[b]).wait()
            pltpu.make_async_copy(c_hbm.at[idx_slice(n)], c_v[b], sem_c[b]).wait()

            @pl.when(n + 2 < NCHUNK)
            def _prefetch_ids():
                pltpu.async_copy(tid_hbm.at[idx_slice(n + 2)], tid_v[b], sem_i[b])
                pltpu.async_copy(mid_hbm.at[idx_slice(n + 2)], mid_v[b], sem_i[b])

            @pl.when((n >= 1) & (n + 1 < NCHUNK))
            def _wait_ids():
                pltpu.make_async_copy(
                    tid_hbm.at[idx_slice(n + 1)], tid_v[nb], sem_i[nb]).wait()
                pltpu.make_async_copy(
                    mid_hbm.at[idx_slice(n + 1)], mid_v[nb], sem_i[nb]).wait()

            @pl.when(n + 1 < NCHUNK)
            def _launch_gathers():
                pltpu.async_copy(et_hbm.at[tid_v[nb]], t_v[nb], sem_t[nb])
                pltpu.async_copy(tt_hbm.at[mid_v[nb]], m_v[nb], sem_m[nb])

            def row_fn(r, rc):
                for j in range(NCOL):
                    sl = pl.ds(j * 16, 16)
                    c_v[b][r, sl] = c_v[b][r, sl] + t_v[b][r, sl] + m_v[b][r, sl]
                return rc

            lax.fori_loop(0, CB, row_fn, 0)

            @pl.when((wid == 0) & (n == 0))
            def _fix_row0():
                for j in range(NCOL):
                    sl = pl.ds(j * 16, 16)
                    c_v[b][0, sl] = c_v[b][0, sl] - m_v[b][0, sl]

            pltpu.async_copy(c_v[b], out_hbm.at[idx_slice(n)], sem_o[b])

            @pl.when(n >= 1)
            def _drain_store():
                pltpu.make_async_copy(
                    c_v[nb], out_hbm.at[idx_slice(n - 1)], sem_o[nb]).wait()

            @pl.when(n + 1 < NCHUNK)
            def _launch_content():
                pltpu.async_copy(c_hbm.at[idx_slice(n + 1)], c_v[nb], sem_c[nb])

        return carry

    lax.fori_loop(0, NCHUNK // 2, pair_fn, 0)

    pltpu.make_async_copy(
        c_v[1], out_hbm.at[idx_slice(NCHUNK - 1)], sem_o[1]).wait()


@jax.jit
def _encode(content_embeddings, type_ids, time_ids, event_table, time_table):
    mesh = plsc.VectorSubcoreMesh(core_axis_name="c", subcore_axis_name="s")
    run = functools.partial(
        pl.kernel,
        mesh=mesh,
        out_type=jax.ShapeDtypeStruct((L, H), jnp.float32),
        scratch_types=[
            pltpu.VMEM((CB,), jnp.int32),
            pltpu.VMEM((CB,), jnp.int32),
            pltpu.VMEM((CB,), jnp.int32),
            pltpu.VMEM((CB,), jnp.int32),
            pltpu.VMEM((CB, H), jnp.float32),
            pltpu.VMEM((CB, H), jnp.float32),
            pltpu.VMEM((CB, H), jnp.float32),
            pltpu.VMEM((CB, H), jnp.float32),
            pltpu.VMEM((CB, H), jnp.float32),
            pltpu.VMEM((CB, H), jnp.float32),
        ] + [pltpu.SemaphoreType.DMA] * 10,
    )(_body)
    return run(content_embeddings, type_ids, time_ids, event_table, time_table)


def kernel(content_embeddings, type_ids, time_ids, event_table, time_table):
    return _encode(
        content_embeddings,
        type_ids.astype(jnp.int32),
        time_ids.astype(jnp.int32),
        event_table,
        time_table,
    )


# speedup vs baseline: 7.9857x; 2.6934x over previous
import functools

import jax
import jax.numpy as jnp
from jax import lax
from jax.experimental import pallas as pl
from jax.experimental.pallas import tpu as pltpu
from jax.experimental.pallas import tpu_sc as plsc

L = 204800
H = 128
NTYPES = 32
NTIME = 1024

NC = 2
NS = 16
NW = NC * NS
RPW = L // NW
CB = 128
NCHUNK = RPW // CB
NCOL = H // 16


def _body(c_hbm, tid_hbm, mid_hbm, et_hbm, tt_hbm, out_hbm,
          tid0, tid1, mid0, mid1, cv0, cv1, tv0, tv1, mv0, mv1,
          et_sp, tt_sp,
          st0, st1, sm0, sm1, sc0, sc1, so0, so1, si0, si1):
    tid_v = (tid0, tid1)
    mid_v = (mid0, mid1)
    c_v = (cv0, cv1)
    t_v = (tv0, tv1)
    m_v = (mv0, mv1)
    sem_t = (st0, st1)
    sem_m = (sm0, sm1)
    sem_c = (sc0, sc1)
    sem_o = (so0, so1)
    sem_i = (si0, si1)

    cid = lax.axis_index("c")
    sid = lax.axis_index("s")
    wid = sid * NC + cid
    base0 = wid * RPW

    def idx_slice(n):
        return pl.ds(base0 + n * CB, CB)

    @pl.when(sid == 0)
    def _stage_tables():
        pltpu.sync_copy(et_hbm, et_sp)
        pltpu.sync_copy(tt_hbm, tt_sp)

    plsc.subcore_barrier()

    pltpu.sync_copy(tid_hbm.at[idx_slice(0)], tid_v[0])
    pltpu.sync_copy(mid_hbm.at[idx_slice(0)], mid_v[0])
    pltpu.sync_copy(tid_hbm.at[idx_slice(1)], tid_v[1])
    pltpu.sync_copy(mid_hbm.at[idx_slice(1)], mid_v[1])
    pltpu.async_copy(et_sp.at[tid_v[0]], t_v[0], sem_t[0])
    pltpu.async_copy(tt_sp.at[mid_v[0]], m_v[0], sem_m[0])
    pltpu.async_copy(c_hbm.at[idx_slice(0)], c_v[0], sem_c[0])

    def pair_fn(g, carry):
        for b in (0, 1):
            nb = 1 - b
            n = 2 * g + b

            pltpu.make_async_copy(et_sp.at[tid_v[b]], t_v[b], sem_t[b]).wait()
            pltpu.make_async_copy(tt_sp.at[mid_v[b]], m_v[b], sem_m[b]).wait()
            pltpu.make_async_copy(c_hbm.at[idx_slice(n)], c_v[b], sem_c[b]).wait()

            @pl.when(n + 2 < NCHUNK)
            def _prefetch_ids():
                pltpu.async_copy(tid_hbm.at[idx_slice(n + 2)], tid_v[b], sem_i[b])
                pltpu.async_copy(mid_hbm.at[idx_slice(n + 2)], mid_v[b], sem_i[b])

            @pl.when((n >= 1) & (n + 1 < NCHUNK))
            def _wait_ids():
                pltpu.make_async_copy(
                    tid_hbm.at[idx_slice(n + 1)], tid_v[nb], sem_i[nb]).wait()
                pltpu.make_async_copy(
                    mid_hbm.at[idx_slice(n + 1)], mid_v[nb], sem_i[nb]).wait()

            @pl.when(n + 1 < NCHUNK)
            def _launch_gathers():
                pltpu.async_copy(et_sp.at[tid_v[nb]], t_v[nb], sem_t[nb])
                pltpu.async_copy(tt_sp.at[mid_v[nb]], m_v[nb], sem_m[nb])

            def row_fn(r, rc):
                for j in range(NCOL):
                    sl = pl.ds(j * 16, 16)
                    c_v[b][r, sl] = c_v[b][r, sl] + t_v[b][r, sl] + m_v[b][r, sl]
                return rc

            lax.fori_loop(0, CB, row_fn, 0)

            @pl.when((wid == 0) & (n == 0))
            def _fix_row0():
                for j in range(NCOL):
                    sl = pl.ds(j * 16, 16)
                    c_v[b][0, sl] = c_v[b][0, sl] - m_v[b][0, sl]

            pltpu.async_copy(c_v[b], out_hbm.at[idx_slice(n)], sem_o[b])

            @pl.when(n >= 1)
            def _drain_store():
                pltpu.make_async_copy(
                    c_v[nb], out_hbm.at[idx_slice(n - 1)], sem_o[nb]).wait()

            @pl.when(n + 1 < NCHUNK)
            def _launch_content():
                pltpu.async_copy(c_hbm.at[idx_slice(n + 1)], c_v[nb], sem_c[nb])

        return carry

    lax.fori_loop(0, NCHUNK // 2, pair_fn, 0)

    pltpu.make_async_copy(
        c_v[1], out_hbm.at[idx_slice(NCHUNK - 1)], sem_o[1]).wait()


@jax.jit
def _encode(content_embeddings, type_ids, time_ids, event_table, time_table):
    mesh = plsc.VectorSubcoreMesh(core_axis_name="c", subcore_axis_name="s")
    run = functools.partial(
        pl.kernel,
        mesh=mesh,
        out_type=jax.ShapeDtypeStruct((L, H), jnp.float32),
        scratch_types=[
            pltpu.VMEM((CB,), jnp.int32),
            pltpu.VMEM((CB,), jnp.int32),
            pltpu.VMEM((CB,), jnp.int32),
            pltpu.VMEM((CB,), jnp.int32),
            pltpu.VMEM((CB, H), jnp.float32),
            pltpu.VMEM((CB, H), jnp.float32),
            pltpu.VMEM((CB, H), jnp.float32),
            pltpu.VMEM((CB, H), jnp.float32),
            pltpu.VMEM((CB, H), jnp.float32),
            pltpu.VMEM((CB, H), jnp.float32),
            pltpu.VMEM_SHARED((NTYPES, H), jnp.float32),
            pltpu.VMEM_SHARED((NTIME, H), jnp.float32),
        ] + [pltpu.SemaphoreType.DMA] * 10,
    )(_body)
    return run(content_embeddings, type_ids, time_ids, event_table, time_table)


def kernel(content_embeddings, type_ids, time_ids, event_table, time_table):
    return _encode(
        content_embeddings,
        type_ids.astype(jnp.int32),
        time_ids.astype(jnp.int32),
        event_table,
        time_table,
    )


# speedup vs baseline: 8.7932x; 1.1011x over previous
import functools

import jax
import jax.numpy as jnp
from jax import lax
from jax.experimental import pallas as pl
from jax.experimental.pallas import tpu as pltpu
from jax.experimental.pallas import tpu_sc as plsc

L = 204800
H = 128
NTYPES = 32
NTIME = 1024

NC = 2
NS = 16
NW = NC * NS
RPW = L // NW
CB = 128
NCHUNK = RPW // CB
NCOL = H // 16


def _body(c_hbm, tid_hbm, mid_hbm, et_hbm, tt_hbm, out_hbm,
          tid0, tid1, mid0, mid1, cv0, cv1, fix_v,
          et_sp, tt_sp,
          st0, st1, sm0, sm1, sc0, sc1, so0, so1, si0, si1):
    tid_v = (tid0, tid1)
    mid_v = (mid0, mid1)
    c_v = (cv0, cv1)
    sem_t = (st0, st1)
    sem_m = (sm0, sm1)
    sem_c = (sc0, sc1)
    sem_o = (so0, so1)
    sem_i = (si0, si1)

    cid = lax.axis_index("c")
    sid = lax.axis_index("s")
    wid = sid * NC + cid
    base0 = wid * RPW

    def idx_slice(n):
        return pl.ds(base0 + n * CB, CB)

    @pl.when(sid == 0)
    def _stage_tables():
        pltpu.sync_copy(et_hbm, et_sp)
        pltpu.sync_copy(tt_hbm, tt_sp)

    plsc.subcore_barrier()

    pltpu.sync_copy(tid_hbm.at[idx_slice(0)], tid_v[0])
    pltpu.sync_copy(mid_hbm.at[idx_slice(0)], mid_v[0])
    pltpu.sync_copy(tid_hbm.at[idx_slice(1)], tid_v[1])
    pltpu.sync_copy(mid_hbm.at[idx_slice(1)], mid_v[1])
    pltpu.async_copy(c_hbm.at[idx_slice(0)], c_v[0], sem_c[0])

    def pair_fn(g, carry):
        for b in (0, 1):
            nb = 1 - b
            n = 2 * g + b

            pltpu.make_async_copy(c_hbm.at[idx_slice(n)], c_v[b], sem_c[b]).wait()

            @pl.when(n >= 2)
            def _wait_ids():
                pltpu.make_async_copy(
                    tid_hbm.at[idx_slice(n)], tid_v[b], sem_i[b]).wait()
                pltpu.make_async_copy(
                    mid_hbm.at[idx_slice(n)], mid_v[b], sem_i[b]).wait()

            ga = pltpu.async_copy(et_sp.at[tid_v[b]], c_v[b], sem_t[b], add=True)
            gb = pltpu.async_copy(tt_sp.at[mid_v[b]], c_v[b], sem_m[b], add=True)
            ga.wait()
            gb.wait()

            @pl.when((wid == 0) & (n == 0))
            def _fix_row0():
                mvec = mid_v[b][pl.ds(0, 16)]
                pltpu.sync_copy(tt_sp.at[pl.ds(mvec[0], 1)], fix_v)
                for j in range(NCOL):
                    sl = pl.ds(j * 16, 16)
                    c_v[b][0, sl] = c_v[b][0, sl] - fix_v[0, sl]

            @pl.when(n + 2 < NCHUNK)
            def _prefetch_ids():
                pltpu.async_copy(tid_hbm.at[idx_slice(n + 2)], tid_v[b], sem_i[b])
                pltpu.async_copy(mid_hbm.at[idx_slice(n + 2)], mid_v[b], sem_i[b])

            pltpu.async_copy(c_v[b], out_hbm.at[idx_slice(n)], sem_o[b])

            @pl.when(n >= 1)
            def _drain_store():
                pltpu.make_async_copy(
                    c_v[nb], out_hbm.at[idx_slice(n - 1)], sem_o[nb]).wait()

            @pl.when(n + 1 < NCHUNK)
            def _launch_content():
                pltpu.async_copy(c_hbm.at[idx_slice(n + 1)], c_v[nb], sem_c[nb])

        return carry

    lax.fori_loop(0, NCHUNK // 2, pair_fn, 0)

    pltpu.make_async_copy(
        c_v[1], out_hbm.at[idx_slice(NCHUNK - 1)], sem_o[1]).wait()


@jax.jit
def _encode(content_embeddings, type_ids, time_ids, event_table, time_table):
    mesh = plsc.VectorSubcoreMesh(core_axis_name="c", subcore_axis_name="s")
    run = functools.partial(
        pl.kernel,
        mesh=mesh,
        out_type=jax.ShapeDtypeStruct((L, H), jnp.float32),
        scratch_types=[
            pltpu.VMEM((CB,), jnp.int32),
            pltpu.VMEM((CB,), jnp.int32),
            pltpu.VMEM((CB,), jnp.int32),
            pltpu.VMEM((CB,), jnp.int32),
            pltpu.VMEM((CB, H), jnp.float32),
            pltpu.VMEM((CB, H), jnp.float32),
            pltpu.VMEM((1, H), jnp.float32),
            pltpu.VMEM_SHARED((NTYPES, H), jnp.float32),
            pltpu.VMEM_SHARED((NTIME, H), jnp.float32),
        ] + [pltpu.SemaphoreType.DMA] * 10,
    )(_body)
    return run(content_embeddings, type_ids, time_ids, event_table, time_table)


def kernel(content_embeddings, type_ids, time_ids, event_table, time_table):
    return _encode(
        content_embeddings,
        type_ids.astype(jnp.int32),
        time_ids.astype(jnp.int32),
        event_table,
        time_table,
    )


# speedup vs baseline: 12.7387x; 1.4487x over previous
import functools

import jax
import jax.numpy as jnp
from jax import lax
from jax.experimental import pallas as pl
from jax.experimental.pallas import tpu as pltpu
from jax.experimental.pallas import tpu_sc as plsc

L = 204800
H = 128
NTYPES = 32
NTIME = 1024

NC = 2
NS = 16
NW = NC * NS
RPW = L // NW
CB = 256
SG = 128
NCHUNK = RPW // CB
NCOL = H // 16
NB = 3
NTRIPLE = NCHUNK // NB


def _body(c_hbm, tid_hbm, mid_hbm, et_hbm, tt_hbm, out_hbm,
          tid0, tid1, tid2, mid0, mid1, mid2, cv0, cv1, cv2, fix_v,
          et_sp, tt_sp,
          st0, st1, st2, sm0, sm1, sm2, sc0, sc1, sc2,
          so0, so1, so2, si0, si1, si2):
    tid_v = (tid0, tid1, tid2)
    mid_v = (mid0, mid1, mid2)
    c_v = (cv0, cv1, cv2)
    sem_t = (st0, st1, st2)
    sem_m = (sm0, sm1, sm2)
    sem_c = (sc0, sc1, sc2)
    sem_o = (so0, so1, so2)
    sem_i = (si0, si1, si2)

    cid = lax.axis_index("c")
    sid = lax.axis_index("s")
    wid = sid * NC + cid
    base0 = wid * RPW

    def idx_slice(n):
        return pl.ds(base0 + n * CB, CB)

    def issue_gadds(n, b, add):
        ds = [
            (et_sp, tid_v[b], sem_t[b], 0),
            (et_sp, tid_v[b], sem_t[b], SG),
            (tt_sp, mid_v[b], sem_m[b], 0),
            (tt_sp, mid_v[b], sem_m[b], SG),
        ]
        out = []
        for table, ids, sem, off in ds:
            out.append(pltpu.make_async_copy(
                table.at[ids.at[pl.ds(off, SG)]],
                c_v[b].at[pl.ds(off, SG)],
                sem))
            if add:
                pltpu.async_copy(
                    table.at[ids.at[pl.ds(off, SG)]],
                    c_v[b].at[pl.ds(off, SG)],
                    sem, add=True)
        return out

    def wait_gadds(n, b):
        for d in issue_gadds(n, b, add=False):
            d.wait()

    @pl.when(sid == 0)
    def _stage_tables():
        pltpu.sync_copy(et_hbm, et_sp)
        pltpu.sync_copy(tt_hbm, tt_sp)

    plsc.subcore_barrier()

    for k in range(NB):
        pltpu.sync_copy(tid_hbm.at[idx_slice(k)], tid_v[k])
        pltpu.sync_copy(mid_hbm.at[idx_slice(k)], mid_v[k])
    pltpu.async_copy(c_hbm.at[idx_slice(0)], c_v[0], sem_c[0])

    def stage(n, b, bp, bpp):
        pltpu.make_async_copy(c_hbm.at[idx_slice(n)], c_v[b], sem_c[b]).wait()

        @pl.when(n >= NB)
        def _wait_ids():
            pltpu.make_async_copy(
                tid_hbm.at[idx_slice(n)], tid_v[b], sem_i[b]).wait()
            pltpu.make_async_copy(
                mid_hbm.at[idx_slice(n)], mid_v[b], sem_i[b]).wait()

        issue_gadds(n, b, add=True)

        @pl.when(n >= 1)
        def _store_prev():
            wait_gadds(n - 1, bp)

            @pl.when((wid == 0) & (n == 1))
            def _fix_row0():
                mvec = mid_v[0][pl.ds(0, 16)]
                pltpu.sync_copy(tt_sp.at[pl.ds(mvec[0], 1)], fix_v)
                for j in range(NCOL):
                    sl = pl.ds(j * 16, 16)
                    c_v[0][0, sl] = c_v[0][0, sl] - fix_v[0, sl]

            pltpu.async_copy(c_v[bp], out_hbm.at[idx_slice(n - 1)], sem_o[bp])

        @pl.when(n >= 2)
        def _drain_store():
            pltpu.make_async_copy(
                c_v[bpp], out_hbm.at[idx_slice(n - 2)], sem_o[bpp]).wait()

        @pl.when(n + 1 < NCHUNK)
        def _launch_content():
            pltpu.async_copy(c_hbm.at[idx_slice(n + 1)], c_v[bpp], sem_c[bpp])

        @pl.when(n + 2 < NCHUNK)
        def _prefetch_ids():
            pltpu.async_copy(tid_hbm.at[idx_slice(n + 2)], tid_v[bp], sem_i[bp])
            pltpu.async_copy(mid_hbm.at[idx_slice(n + 2)], mid_v[bp], sem_i[bp])

    def triple_fn(g, carry):
        for b in range(NB):
            n = NB * g + b
            stage(n, b, (b - 1) % NB, (b - 2) % NB)
        return carry

    lax.fori_loop(0, NTRIPLE, triple_fn, 0)

    nl = NB * NTRIPLE
    stage(nl, nl % NB, (nl - 1) % NB, (nl - 2) % NB)
    wait_gadds(nl, nl % NB)
    pltpu.async_copy(c_v[nl % NB], out_hbm.at[idx_slice(nl)], sem_o[nl % NB])
    pltpu.make_async_copy(
        c_v[(nl - 1) % NB], out_hbm.at[idx_slice(nl - 1)],
        sem_o[(nl - 1) % NB]).wait()
    pltpu.make_async_copy(
        c_v[nl % NB], out_hbm.at[idx_slice(nl)], sem_o[nl % NB]).wait()


@jax.jit
def _encode(content_embeddings, type_ids, time_ids, event_table, time_table):
    mesh = plsc.VectorSubcoreMesh(core_axis_name="c", subcore_axis_name="s")
    run = functools.partial(
        pl.kernel,
        mesh=mesh,
        out_type=jax.ShapeDtypeStruct((L, H), jnp.float32),
        scratch_types=[
            pltpu.VMEM((CB,), jnp.int32),
            pltpu.VMEM((CB,), jnp.int32),
            pltpu.VMEM((CB,), jnp.int32),
            pltpu.VMEM((CB,), jnp.int32),
            pltpu.VMEM((CB,), jnp.int32),
            pltpu.VMEM((CB,), jnp.int32),
            pltpu.VMEM((CB, H), jnp.float32),
            pltpu.VMEM((CB, H), jnp.float32),
            pltpu.VMEM((CB, H), jnp.float32),
            pltpu.VMEM((1, H), jnp.float32),
            pltpu.VMEM_SHARED((NTYPES, H), jnp.float32),
            pltpu.VMEM_SHARED((NTIME, H), jnp.float32),
        ] + [pltpu.SemaphoreType.DMA] * 15,
    )(_body)
    return run(content_embeddings, type_ids, time_ids, event_table, time_table)


def kernel(content_embeddings, type_ids, time_ids, event_table, time_table):
    return _encode(
        content_embeddings,
        type_ids.astype(jnp.int32),
        time_ids.astype(jnp.int32),
        event_table,
        time_table,
    )


# speedup vs baseline: 12.9467x; 1.0163x over previous
import functools

import jax
import jax.numpy as jnp
from jax import lax
from jax.experimental import pallas as pl
from jax.experimental.pallas import tpu as pltpu
from jax.experimental.pallas import tpu_sc as plsc

L = 204800
H = 128
NTYPES = 32
NTIME = 1024

NC = 2
NS = 16
NW = NC * NS
RPW = L // NW
CB = 256
SG = 128
NCHUNK = RPW // CB
NCOL = H // 16
NB = 3
NTRIPLE = NCHUNK // NB


def _body(c_hbm, tid_hbm, mid_hbm, et_hbm, tt_hbm, out_hbm,
          tid_all, mid_all, cv0, cv1, cv2, fix_v,
          et_sp, tt_sp,
          st0, st1, st2, sm0, sm1, sm2, sc0, sc1, sc2, so0, so1, so2):
    c_v = (cv0, cv1, cv2)
    sem_t = (st0, st1, st2)
    sem_m = (sm0, sm1, sm2)
    sem_c = (sc0, sc1, sc2)
    sem_o = (so0, so1, so2)

    cid = lax.axis_index("c")
    sid = lax.axis_index("s")
    wid = sid * NC + cid
    base0 = wid * RPW

    def idx_slice(n):
        return pl.ds(base0 + n * CB, CB)

    def gadd_descs(n, b):
        out = []
        for table, ids, sem in ((et_sp, tid_all, sem_t[b]),
                                (tt_sp, mid_all, sem_m[b])):
            for off in (0, SG):
                out.append((
                    table.at[ids.at[pl.ds(n * CB + off, SG)]],
                    c_v[b].at[pl.ds(off, SG)],
                    sem))
        return out

    def issue_gadds(n, b):
        for src, dst, sem in gadd_descs(n, b):
            pltpu.async_copy(src, dst, sem, add=True)

    def wait_gadds(n, b):
        for src, dst, sem in gadd_descs(n, b):
            pltpu.make_async_copy(src, dst, sem).wait()

    @pl.when(sid == 0)
    def _stage_tables():
        pltpu.sync_copy(et_hbm, et_sp)
        pltpu.sync_copy(tt_hbm, tt_sp)

    pltpu.sync_copy(tid_hbm.at[pl.ds(base0, RPW)], tid_all)
    pltpu.sync_copy(mid_hbm.at[pl.ds(base0, RPW)], mid_all)
    plsc.subcore_barrier()

    pltpu.async_copy(c_hbm.at[idx_slice(0)], c_v[0], sem_c[0])

    def stage(n, b, bp, bpp):
        pltpu.make_async_copy(c_hbm.at[idx_slice(n)], c_v[b], sem_c[b]).wait()
        issue_gadds(n, b)

        @pl.when(n >= 1)
        def _store_prev():
            wait_gadds(n - 1, bp)

            @pl.when((wid == 0) & (n == 1))
            def _fix_row0():
                mvec = mid_all[pl.ds(0, 16)]
                pltpu.sync_copy(tt_sp.at[pl.ds(mvec[0], 1)], fix_v)
                for j in range(NCOL):
                    sl = pl.ds(j * 16, 16)
                    c_v[0][0, sl] = c_v[0][0, sl] - fix_v[0, sl]

            pltpu.async_copy(c_v[bp], out_hbm.at[idx_slice(n - 1)], sem_o[bp])

        @pl.when(n >= 2)
        def _drain_store():
            pltpu.make_async_copy(
                c_v[bpp], out_hbm.at[idx_slice(n - 2)], sem_o[bpp]).wait()

        @pl.when(n + 1 < NCHUNK)
        def _launch_content():
            pltpu.async_copy(c_hbm.at[idx_slice(n + 1)], c_v[bpp], sem_c[bpp])

    def triple_fn(g, carry):
        for b in range(NB):
            n = NB * g + b
            stage(n, b, (b - 1) % NB, (b - 2) % NB)
        return carry

    lax.fori_loop(0, NTRIPLE, triple_fn, 0)

    nl = NB * NTRIPLE
    stage(nl, nl % NB, (nl - 1) % NB, (nl - 2) % NB)
    wait_gadds(nl, nl % NB)
    pltpu.async_copy(c_v[nl % NB], out_hbm.at[idx_slice(nl)], sem_o[nl % NB])
    pltpu.make_async_copy(
        c_v[(nl - 1) % NB], out_hbm.at[idx_slice(nl - 1)],
        sem_o[(nl - 1) % NB]).wait()
    pltpu.make_async_copy(
        c_v[nl % NB], out_hbm.at[idx_slice(nl)], sem_o[nl % NB]).wait()


@jax.jit
def _encode(content_embeddings, type_ids, time_ids, event_table, time_table):
    mesh = plsc.VectorSubcoreMesh(core_axis_name="c", subcore_axis_name="s")
    run = functools.partial(
        pl.kernel,
        mesh=mesh,
        out_type=jax.ShapeDtypeStruct((L, H), jnp.float32),
        scratch_types=[
            pltpu.VMEM((RPW,), jnp.int32),
            pltpu.VMEM((RPW,), jnp.int32),
            pltpu.VMEM((CB, H), jnp.float32),
            pltpu.VMEM((CB, H), jnp.float32),
            pltpu.VMEM((CB, H), jnp.float32),
            pltpu.VMEM((1, H), jnp.float32),
            pltpu.VMEM_SHARED((NTYPES, H), jnp.float32),
            pltpu.VMEM_SHARED((NTIME, H), jnp.float32),
        ] + [pltpu.SemaphoreType.DMA] * 12,
    )(_body)
    return run(content_embeddings, type_ids, time_ids, event_table, time_table)


def kernel(content_embeddings, type_ids, time_ids, event_table, time_table):
    return _encode(
        content_embeddings,
        type_ids.astype(jnp.int32),
        time_ids.astype(jnp.int32),
        event_table,
        time_table,
    )
